# initial kernel scaffold (unmeasured)
import jax
import jax.numpy as jnp
from jax import lax
from jax.experimental import pallas as pl
from jax.experimental.pallas import tpu as pltpu

N_DEV = 4
R = 2048
D = 2048


def kernel(partial, resid, gamma):
    M = resid.shape[0]
    assert partial.shape == (1, M, D) and M == N_DEV * R

    gamma2 = gamma.reshape(1, D)

    def body(partial_ref, resid_ref, gamma_ref, out_ref,
             lstage, sendb, recvb, outstage,
             load_sem, store_sem, send_sems, recv_sems):
        i = lax.axis_index("i")
        left = (i + N_DEV - 1) % N_DEV
        right = (i + 1) % N_DEV

        barrier_sem = pltpu.get_barrier_semaphore()
        for nbr in (left, right):
            pl.semaphore_signal(
                barrier_sem, inc=1,
                device_id=(nbr,), device_id_type=pl.DeviceIdType.MESH,
            )
        pl.semaphore_wait(barrier_sem, 2)

        def load_partial(c, buf):
            cp = pltpu.make_async_copy(
                partial_ref.at[0, pl.ds(c * R, R), :], buf, load_sem)
            cp.start()
            cp.wait()

        def hop(s, src):
            recv_slot = s % 4
            rdma = pltpu.make_async_remote_copy(
                src_ref=src,
                dst_ref=recvb.at[recv_slot],
                send_sem=send_sems.at[s % 2],
                recv_sem=recv_sems.at[recv_slot],
                device_id=(right,),
                device_id_type=pl.DeviceIdType.MESH,
            )
            rdma.start()
            return rdma, recv_slot

        load_partial(i, lstage)
        sendb[0, :, :] = lstage[...].astype(jnp.bfloat16)

        for s in range(N_DEV - 1):
            rdma, recv_slot = hop(s, sendb.at[s % 2])
            c_recv = (i + N_DEV - s - 1) % N_DEV
            load_partial(c_recv, lstage)
            rdma.wait()
            acc = recvb[recv_slot, :, :].astype(jnp.float32) + lstage[...]
            if s < N_DEV - 2:
                sendb[(s + 1) % 2, :, :] = acc.astype(jnp.bfloat16)
            else:
                f = (i + 1) % N_DEV
                outstage[...] = acc
                cp = pltpu.make_async_copy(
                    resid_ref.at[pl.ds(f * R, R), :], lstage, load_sem)
                cp.start()
                cp.wait()
                y = outstage[...] + lstage[...]
                rms = jnp.sqrt(
                    jnp.mean(y * y, axis=-1, keepdims=True) + 1e-6)
                o = y / rms * gamma_ref[...]
                outstage[...] = o
                st = pltpu.make_async_copy(
                    outstage, out_ref.at[pl.ds(f * R, R), :], store_sem)
                st.start()
                st.wait()
                sendb[1, :, :] = o.astype(jnp.bfloat16)

        for h in range(N_DEV - 1):
            s = (N_DEV - 1) + h
            src = sendb.at[1] if h == 0 else recvb.at[(s - 1) % 4]
            rdma, recv_slot = hop(s, src)
            rdma.wait()
            g = (i + N_DEV - h) % N_DEV
            outstage[...] = recvb[recv_slot, :, :].astype(jnp.float32)
            st = pltpu.make_async_copy(
                outstage, out_ref.at[pl.ds(g * R, R), :], store_sem)
            st.start()
            st.wait()

    return pl.pallas_call(
        body,
        out_shape=jax.ShapeDtypeStruct((M, D), jnp.float32),
        in_specs=[
            pl.BlockSpec(memory_space=pltpu.ANY),
            pl.BlockSpec(memory_space=pltpu.ANY),
            pl.BlockSpec(memory_space=pltpu.VMEM),
        ],
        out_specs=pl.BlockSpec(memory_space=pltpu.ANY),
        scratch_shapes=[
            pltpu.VMEM((R, D), jnp.float32),
            pltpu.VMEM((2, R, D), jnp.bfloat16),
            pltpu.VMEM((4, R, D), jnp.bfloat16),
            pltpu.VMEM((R, D), jnp.float32),
            pltpu.SemaphoreType.DMA,
            pltpu.SemaphoreType.DMA,
            pltpu.SemaphoreType.DMA((2,)),
            pltpu.SemaphoreType.DMA((4,)),
        ],
        compiler_params=pltpu.CompilerParams(
            collective_id=0,
            vmem_limit_bytes=120 * 1024 * 1024,
        ),
    )(partial, resid, gamma2)


# baseline (device time: 661199 ns/iter reference)
import jax
import jax.numpy as jnp
from jax import lax
from jax.experimental import pallas as pl
from jax.experimental.pallas import tpu as pltpu

N_DEV = 4
R = 2048
R_SUB = 1024
P = R // R_SUB
D = 2048


def kernel(partial, resid, gamma):
    M = resid.shape[0]
    assert partial.shape == (1, M, D) and M == N_DEV * R

    gamma2 = gamma.reshape(1, D)

    def body(partial_ref, resid_ref, gamma_ref, out_ref,
             lstage, sendb, recvb, outstage,
             load_sem, store_sem, send_sems, recv_sems):
        i = lax.axis_index("i")
        left = (i + N_DEV - 1) % N_DEV
        right = (i + 1) % N_DEV

        barrier_sem = pltpu.get_barrier_semaphore()
        for nbr in (left, right):
            pl.semaphore_signal(
                barrier_sem, inc=1,
                device_id=(nbr,), device_id_type=pl.DeviceIdType.MESH,
            )
        pl.semaphore_wait(barrier_sem, 2)

        def hop(t, src):
            recv_slot = t % 4
            rdma = pltpu.make_async_remote_copy(
                src_ref=src,
                dst_ref=recvb.at[recv_slot],
                send_sem=send_sems.at[t % 2],
                recv_sem=recv_sems.at[recv_slot],
                device_id=(right,),
                device_id_type=pl.DeviceIdType.MESH,
            )
            rdma.start()
            return rdma, recv_slot

        for p in range(P):
            def load(ref, c, buf):
                cp = pltpu.make_async_copy(
                    ref.at[pl.ds(c * R + p * R_SUB, R_SUB), :], buf,
                    load_sem)
                cp.start()
                cp.wait()

            load(partial_ref.at[0], i, lstage)
            sendb[0, :, :] = lstage[...].astype(jnp.bfloat16)

            for s in range(N_DEV - 1):
                t = p * 2 * (N_DEV - 1) + s
                rdma, recv_slot = hop(t, sendb.at[s % 2])
                c_recv = (i + N_DEV - s - 1) % N_DEV
                load(partial_ref.at[0], c_recv, lstage)
                rdma.wait()
                acc = (recvb[recv_slot, :, :].astype(jnp.float32)
                       + lstage[...])
                if s < N_DEV - 2:
                    sendb[(s + 1) % 2, :, :] = acc.astype(jnp.bfloat16)
                else:
                    f = (i + 1) % N_DEV
                    outstage[...] = acc
                    load(resid_ref, f, lstage)
                    y = outstage[...] + lstage[...]
                    rms = jnp.sqrt(
                        jnp.mean(y * y, axis=-1, keepdims=True) + 1e-6)
                    o = y / rms * gamma_ref[...]
                    outstage[...] = o
                    st = pltpu.make_async_copy(
                        outstage,
                        out_ref.at[pl.ds(f * R + p * R_SUB, R_SUB), :],
                        store_sem)
                    st.start()
                    st.wait()
                    sendb[1, :, :] = o.astype(jnp.bfloat16)

            for h in range(N_DEV - 1):
                t = p * 2 * (N_DEV - 1) + (N_DEV - 1) + h
                src = sendb.at[1] if h == 0 else recvb.at[(t - 1) % 4]
                rdma, recv_slot = hop(t, src)
                rdma.wait()
                g = (i + N_DEV - h) % N_DEV
                outstage[...] = recvb[recv_slot, :, :].astype(jnp.float32)
                st = pltpu.make_async_copy(
                    outstage,
                    out_ref.at[pl.ds(g * R + p * R_SUB, R_SUB), :],
                    store_sem)
                st.start()
                st.wait()

    return pl.pallas_call(
        body,
        out_shape=jax.ShapeDtypeStruct((M, D), jnp.float32),
        in_specs=[
            pl.BlockSpec(memory_space=pl.ANY),
            pl.BlockSpec(memory_space=pl.ANY),
            pl.BlockSpec(memory_space=pltpu.VMEM),
        ],
        out_specs=pl.BlockSpec(memory_space=pl.ANY),
        scratch_shapes=[
            pltpu.VMEM((R_SUB, D), jnp.float32),
            pltpu.VMEM((2, R_SUB, D), jnp.bfloat16),
            pltpu.VMEM((4, R_SUB, D), jnp.bfloat16),
            pltpu.VMEM((R_SUB, D), jnp.float32),
            pltpu.SemaphoreType.DMA,
            pltpu.SemaphoreType.DMA,
            pltpu.SemaphoreType.DMA((2,)),
            pltpu.SemaphoreType.DMA((4,)),
        ],
        compiler_params=pltpu.CompilerParams(
            collective_id=0,
            vmem_limit_bytes=60 * 1024 * 1024,
        ),
    )(partial, resid, gamma2)


# device time: 388524 ns/iter; 1.7018x vs baseline; 1.7018x over previous
import jax
import jax.numpy as jnp
from jax import lax
from jax.experimental import pallas as pl
from jax.experimental.pallas import tpu as pltpu

N_DEV = 4
R = 2048
R_SUB = 1024
H = R_SUB // 2
P = R // R_SUB
D = 2048


def kernel(partial, resid, gamma):
    M = resid.shape[0]
    assert partial.shape == (1, M, D) and M == N_DEV * R

    gamma2 = gamma.reshape(1, D)

    def body(partial_ref, resid_ref, gamma_ref, out_ref,
             lstage_a, lstage_b, sendb_a, sendb_b, recvb_a, recvb_b,
             outstage_a, outstage_b,
             load_sems, store_sems,
             send_sems_a, send_sems_b, recv_sems_a, recv_sems_b):
        i = lax.axis_index("i")
        left = (i + N_DEV - 1) % N_DEV
        right = (i + 1) % N_DEV

        barrier_sem = pltpu.get_barrier_semaphore()
        for nbr in (left, right):
            pl.semaphore_signal(
                barrier_sem, inc=1,
                device_id=(nbr,), device_id_type=pl.DeviceIdType.MESH,
            )
        pl.semaphore_wait(barrier_sem, 2)

        rings = [
            dict(off=0, dst=right,
                 c_recv=lambda s: (i + N_DEV - s - 1) % N_DEV,
                 f=(i + 1) % N_DEV,
                 g=lambda h: (i + N_DEV - h) % N_DEV,
                 lstage=lstage_a, sendb=sendb_a, recvb=recvb_a,
                 outstage=outstage_a, load_sem=load_sems.at[0],
                 store_sem=store_sems.at[0], send_sems=send_sems_a,
                 recv_sems=recv_sems_a),
            dict(off=H, dst=left,
                 c_recv=lambda s: (i + s + 1) % N_DEV,
                 f=(i + N_DEV - 1) % N_DEV,
                 g=lambda h: (i + h) % N_DEV,
                 lstage=lstage_b, sendb=sendb_b, recvb=recvb_b,
                 outstage=outstage_b, load_sem=load_sems.at[1],
                 store_sem=store_sems.at[1], send_sems=send_sems_b,
                 recv_sems=recv_sems_b),
        ]

        def start_load(ref, c, p, r):
            cp = pltpu.make_async_copy(
                ref.at[pl.ds(c * R + p * R_SUB + r["off"], H), :],
                r["lstage"], r["load_sem"])
            cp.start()
            return cp

        def start_hop(t, src, r):
            rdma = pltpu.make_async_remote_copy(
                src_ref=src,
                dst_ref=r["recvb"].at[t % 4],
                send_sem=r["send_sems"].at[t % 2],
                recv_sem=r["recv_sems"].at[t % 4],
                device_id=(r["dst"],),
                device_id_type=pl.DeviceIdType.MESH,
            )
            rdma.start()
            return rdma

        def start_store(row0, r):
            st = pltpu.make_async_copy(
                r["outstage"], out_ref.at[pl.ds(row0, H), :],
                r["store_sem"])
            st.start()
            return st

        for p in range(P):
            loads = [start_load(partial_ref.at[0], i, p, r)
                     for r in rings]
            for cp, r in zip(loads, rings):
                cp.wait()
                r["sendb"][0, :, :] = r["lstage"][...].astype(jnp.bfloat16)

            for s in range(N_DEV - 1):
                t = p * 2 * (N_DEV - 1) + s
                rdmas = [start_hop(t, r["sendb"].at[s % 2], r)
                         for r in rings]
                loads = [start_load(partial_ref.at[0], r["c_recv"](s),
                                    p, r)
                         for r in rings]
                for cp in loads:
                    cp.wait()
                for rdma in rdmas:
                    rdma.wait()
                for r in rings:
                    acc = (r["recvb"][t % 4, :, :].astype(jnp.float32)
                           + r["lstage"][...])
                    if s < N_DEV - 2:
                        r["sendb"][(s + 1) % 2, :, :] = (
                            acc.astype(jnp.bfloat16))
                    else:
                        r["outstage"][...] = acc
                if s == N_DEV - 2:
                    loads = [start_load(resid_ref, r["f"], p, r)
                             for r in rings]
                    for cp in loads:
                        cp.wait()
                    for r in rings:
                        y = r["outstage"][...] + r["lstage"][...]
                        rms = jnp.sqrt(
                            jnp.mean(y * y, axis=-1, keepdims=True)
                            + 1e-6)
                        o = y / rms * gamma_ref[...]
                        r["outstage"][...] = o
                        r["sendb"][1, :, :] = o.astype(jnp.bfloat16)
                    stores = [
                        start_store(r["f"] * R + p * R_SUB + r["off"], r)
                        for r in rings]
                    for st in stores:
                        st.wait()

            for h in range(N_DEV - 1):
                t = p * 2 * (N_DEV - 1) + (N_DEV - 1) + h
                rdmas = [
                    start_hop(
                        t,
                        r["sendb"].at[1] if h == 0
                        else r["recvb"].at[(t - 1) % 4],
                        r)
                    for r in rings]
                for rdma in rdmas:
                    rdma.wait()
                for r in rings:
                    r["outstage"][...] = (
                        r["recvb"][t % 4, :, :].astype(jnp.float32))
                stores = [
                    start_store(r["g"](h) * R + p * R_SUB + r["off"], r)
                    for r in rings]
                for st in stores:
                    st.wait()

    return pl.pallas_call(
        body,
        out_shape=jax.ShapeDtypeStruct((M, D), jnp.float32),
        in_specs=[
            pl.BlockSpec(memory_space=pl.ANY),
            pl.BlockSpec(memory_space=pl.ANY),
            pl.BlockSpec(memory_space=pltpu.VMEM),
        ],
        out_specs=pl.BlockSpec(memory_space=pl.ANY),
        scratch_shapes=[
            pltpu.VMEM((H, D), jnp.float32),
            pltpu.VMEM((H, D), jnp.float32),
            pltpu.VMEM((2, H, D), jnp.bfloat16),
            pltpu.VMEM((2, H, D), jnp.bfloat16),
            pltpu.VMEM((4, H, D), jnp.bfloat16),
            pltpu.VMEM((4, H, D), jnp.bfloat16),
            pltpu.VMEM((H, D), jnp.float32),
            pltpu.VMEM((H, D), jnp.float32),
            pltpu.SemaphoreType.DMA((2,)),
            pltpu.SemaphoreType.DMA((2,)),
            pltpu.SemaphoreType.DMA((2,)),
            pltpu.SemaphoreType.DMA((2,)),
            pltpu.SemaphoreType.DMA((4,)),
            pltpu.SemaphoreType.DMA((4,)),
        ],
        compiler_params=pltpu.CompilerParams(
            collective_id=0,
            vmem_limit_bytes=60 * 1024 * 1024,
        ),
    )(partial, resid, gamma2)


# device time: 361485 ns/iter; 1.8291x vs baseline; 1.0748x over previous
import jax
import jax.numpy as jnp
from jax import lax
from jax.experimental import pallas as pl
from jax.experimental.pallas import tpu as pltpu

N_DEV = 4
R = 2048
R_SUB = 1024
H = R_SUB // 2
P = R // R_SUB
D = 2048
NSLOT = 5


def kernel(partial, resid, gamma):
    M = resid.shape[0]
    assert partial.shape == (1, M, D) and M == N_DEV * R

    gamma2 = gamma.reshape(1, D)

    def body(partial_ref, resid_ref, gamma_ref, out_ref,
             lstage_a, lstage_b, sendb_a, sendb_b, recvb_a, recvb_b,
             outstage_a, outstage_b,
             load_sems, store_sems,
             send_sems_a, send_sems_b, recv_sems_a, recv_sems_b):
        i = lax.axis_index("i")
        left = (i + N_DEV - 1) % N_DEV
        right = (i + 1) % N_DEV

        barrier_sem = pltpu.get_barrier_semaphore()
        for nbr in (left, right):
            pl.semaphore_signal(
                barrier_sem, inc=1,
                device_id=(nbr,), device_id_type=pl.DeviceIdType.MESH,
            )
        pl.semaphore_wait(barrier_sem, 2)

        rings = [
            dict(off=0, dst=right,
                 c_recv=lambda s: (i + N_DEV - s - 1) % N_DEV,
                 f=(i + 1) % N_DEV,
                 g=lambda h: (i + N_DEV - h) % N_DEV,
                 lstage=lstage_a, sendb=sendb_a, recvb=recvb_a,
                 outstage=outstage_a, load_sem=load_sems.at[0],
                 store_sems=store_sems.at[0], send_sems=send_sems_a,
                 recv_sems=recv_sems_a, pending=[None, None]),
            dict(off=H, dst=left,
                 c_recv=lambda s: (i + s + 1) % N_DEV,
                 f=(i + N_DEV - 1) % N_DEV,
                 g=lambda h: (i + h) % N_DEV,
                 lstage=lstage_b, sendb=sendb_b, recvb=recvb_b,
                 outstage=outstage_b, load_sem=load_sems.at[1],
                 store_sems=store_sems.at[1], send_sems=send_sems_b,
                 recv_sems=recv_sems_b, pending=[None, None]),
        ]

        def start_load(ref, c, p, r, dst):
            cp = pltpu.make_async_copy(
                ref.at[pl.ds(c * R + p * R_SUB + r["off"], H), :],
                dst, r["load_sem"])
            cp.start()
            return cp

        def start_hop(t, src, r):
            rdma = pltpu.make_async_remote_copy(
                src_ref=src,
                dst_ref=r["recvb"].at[t % NSLOT],
                send_sem=r["send_sems"].at[t % 2],
                recv_sem=r["recv_sems"].at[t % NSLOT],
                device_id=(r["dst"],),
                device_id_type=pl.DeviceIdType.MESH,
            )
            rdma.start()
            return rdma

        def wait_pending(r, slot):
            if r["pending"][slot] is not None:
                r["pending"][slot].wait()
                r["pending"][slot] = None

        def store_from_slot(row0, r, slot):
            wait_pending(r, slot)
            st = pltpu.make_async_copy(
                r["outstage"].at[slot], out_ref.at[pl.ds(row0, H), :],
                r["store_sems"].at[slot])
            st.start()
            r["pending"][slot] = st

        for p in range(P):
            loads = [start_load(partial_ref.at[0], i, p, r, r["lstage"])
                     for r in rings]
            for cp, r in zip(loads, rings):
                cp.wait()
                r["sendb"][0, :, :] = r["lstage"][...].astype(jnp.bfloat16)

            rdma_next = [None, None]
            for s in range(N_DEV - 1):
                t = p * 2 * (N_DEV - 1) + s
                rdmas = [start_hop(t, r["sendb"].at[s % 2], r)
                         for r in rings]
                loads = [start_load(partial_ref.at[0], r["c_recv"](s),
                                    p, r, r["lstage"])
                         for r in rings]
                if s == 0:
                    for r in rings:
                        wait_pending(r, 1)
                    rloads = [start_load(resid_ref, r["f"], p, r,
                                         r["outstage"].at[1])
                              for r in rings]
                for cp in loads:
                    cp.wait()
                if s == 0:
                    for cp in rloads:
                        cp.wait()
                for rdma in rdmas:
                    rdma.wait()
                for k, r in enumerate(rings):
                    acc = (r["recvb"][t % NSLOT, :, :]
                           .astype(jnp.float32) + r["lstage"][...])
                    if s < N_DEV - 2:
                        r["sendb"][(s + 1) % 2, :, :] = (
                            acc.astype(jnp.bfloat16))
                    else:
                        y = acc + r["outstage"][1, :, :]
                        rms = jnp.sqrt(
                            jnp.mean(y * y, axis=-1, keepdims=True)
                            + 1e-6)
                        o = y / rms * gamma_ref[...]
                        r["sendb"][1, :, :] = o.astype(jnp.bfloat16)
                        wait_pending(r, 0)
                        r["outstage"][0, :, :] = o
                if s == N_DEV - 2:
                    rdma_next = [start_hop(t + 1, r["sendb"].at[1], r)
                                 for r in rings]
                    for r in rings:
                        store_from_slot(
                            r["f"] * R + p * R_SUB + r["off"], r, 0)

            for h in range(N_DEV - 1):
                t = p * 2 * (N_DEV - 1) + (N_DEV - 1) + h
                rdmas = rdma_next
                for rdma in rdmas:
                    rdma.wait()
                if h < N_DEV - 2:
                    rdma_next = [
                        start_hop(t + 1, r["recvb"].at[t % NSLOT], r)
                        for r in rings]
                slot = (h + 1) % 2
                for r in rings:
                    wait_pending(r, slot)
                    r["outstage"][slot, :, :] = (
                        r["recvb"][t % NSLOT, :, :].astype(jnp.float32))
                for r in rings:
                    store_from_slot(
                        r["g"](h) * R + p * R_SUB + r["off"], r, slot)

        for r in rings:
            for slot in (0, 1):
                wait_pending(r, slot)

    return pl.pallas_call(
        body,
        out_shape=jax.ShapeDtypeStruct((M, D), jnp.float32),
        in_specs=[
            pl.BlockSpec(memory_space=pl.ANY),
            pl.BlockSpec(memory_space=pl.ANY),
            pl.BlockSpec(memory_space=pltpu.VMEM),
        ],
        out_specs=pl.BlockSpec(memory_space=pl.ANY),
        scratch_shapes=[
            pltpu.VMEM((H, D), jnp.float32),
            pltpu.VMEM((H, D), jnp.float32),
            pltpu.VMEM((2, H, D), jnp.bfloat16),
            pltpu.VMEM((2, H, D), jnp.bfloat16),
            pltpu.VMEM((NSLOT, H, D), jnp.bfloat16),
            pltpu.VMEM((NSLOT, H, D), jnp.bfloat16),
            pltpu.VMEM((2, H, D), jnp.float32),
            pltpu.VMEM((2, H, D), jnp.float32),
            pltpu.SemaphoreType.DMA((2,)),
            pltpu.SemaphoreType.DMA((2, 2)),
            pltpu.SemaphoreType.DMA((2,)),
            pltpu.SemaphoreType.DMA((2,)),
            pltpu.SemaphoreType.DMA((NSLOT,)),
            pltpu.SemaphoreType.DMA((NSLOT,)),
        ],
        compiler_params=pltpu.CompilerParams(
            collective_id=0,
            vmem_limit_bytes=63 * 1024 * 1024,
        ),
    )(partial, resid, gamma2)


# device time: 327693 ns/iter; 2.0177x vs baseline; 1.1031x over previous
import jax
import jax.numpy as jnp
from jax import lax
from jax.experimental import pallas as pl
from jax.experimental.pallas import tpu as pltpu

N_DEV = 4
R = 2048
R_SUB = 1024
NSTR = 4
H = R_SUB // NSTR
P = R // R_SUB
D = 2048
NSLOT = 5


def kernel(partial, resid, gamma):
    M = resid.shape[0]
    assert partial.shape == (1, M, D) and M == N_DEV * R

    gamma2 = gamma.reshape(1, D)

    def body(partial_ref, resid_ref, gamma_ref, out_ref,
             lstage, sendb, recvb, outstage,
             load_sems, resid_sems, store_sems, send_sems, recv_sems):
        i = lax.axis_index("i")
        left = (i + N_DEV - 1) % N_DEV
        right = (i + 1) % N_DEV

        barrier_sem = pltpu.get_barrier_semaphore()
        for nbr in (left, right):
            pl.semaphore_signal(
                barrier_sem, inc=1,
                device_id=(nbr,), device_id_type=pl.DeviceIdType.MESH,
            )
        pl.semaphore_wait(barrier_sem, 2)

        ring_a = dict(dst=right,
                      c_recv=lambda s: (i + N_DEV - s - 1) % N_DEV,
                      f=(i + 1) % N_DEV,
                      g=lambda h: (i + N_DEV - h) % N_DEV)
        ring_b = dict(dst=left,
                      c_recv=lambda s: (i + s + 1) % N_DEV,
                      f=(i + N_DEV - 1) % N_DEV,
                      g=lambda h: (i + h) % N_DEV)
        streams = [
            dict(k=0, off=0 * H, **ring_a, pending=[None, None]),
            dict(k=1, off=2 * H, **ring_b, pending=[None, None]),
            dict(k=2, off=1 * H, **ring_a, pending=[None, None]),
            dict(k=3, off=3 * H, **ring_b, pending=[None, None]),
        ]

        def start_load(ref, c, p, st, dst, sem):
            cp = pltpu.make_async_copy(
                ref.at[pl.ds(c * R + p * R_SUB + st["off"], H), :],
                dst, sem)
            cp.start()
            return cp

        def start_hop(t, src, st):
            k = st["k"]
            rdma = pltpu.make_async_remote_copy(
                src_ref=src,
                dst_ref=recvb.at[k, t % NSLOT],
                send_sem=send_sems.at[k * 2 + t % 2],
                recv_sem=recv_sems.at[k * NSLOT + t % NSLOT],
                device_id=(st["dst"],),
                device_id_type=pl.DeviceIdType.MESH,
            )
            rdma.start()
            return rdma

        def wait_pending(st, slot):
            if st["pending"][slot] is not None:
                st["pending"][slot].wait()
                st["pending"][slot] = None

        def store_from_slot(row0, st, slot):
            wait_pending(st, slot)
            k = st["k"]
            s_ = pltpu.make_async_copy(
                outstage.at[k, slot], out_ref.at[pl.ds(row0, H), :],
                store_sems.at[k * 2 + slot])
            s_.start()
            st["pending"][slot] = s_

        rdma_cur = [None] * NSTR
        loads = [None] * NSTR
        for p in range(P):
            t0 = p * 2 * (N_DEV - 1)

            for st in streams:
                k = st["k"]
                loads[k] = start_load(partial_ref.at[0], i, p, st,
                                      lstage.at[k], load_sems.at[k])
            rloads = [None] * NSTR
            for st in streams:
                k = st["k"]
                loads[k].wait()
                sendb[k, 0, :, :] = lstage[k, :, :].astype(jnp.bfloat16)
                rdma_cur[k] = start_hop(t0, sendb.at[k, 0], st)
                loads[k] = start_load(partial_ref.at[0],
                                      st["c_recv"](0), p, st,
                                      lstage.at[k], load_sems.at[k])
                wait_pending(st, 1)
                rloads[k] = start_load(resid_ref, st["f"], p, st,
                                       outstage.at[k, 1],
                                       resid_sems.at[k])

            for s in range(N_DEV - 1):
                t = t0 + s
                for st in streams:
                    k = st["k"]
                    loads[k].wait()
                    rdma_cur[k].wait()
                    acc = (recvb[k, t % NSLOT, :, :]
                           .astype(jnp.float32) + lstage[k, :, :])
                    if s < N_DEV - 2:
                        sendb[k, (s + 1) % 2, :, :] = (
                            acc.astype(jnp.bfloat16))
                        rdma_cur[k] = start_hop(
                            t + 1, sendb.at[k, (s + 1) % 2], st)
                        loads[k] = start_load(
                            partial_ref.at[0], st["c_recv"](s + 1), p,
                            st, lstage.at[k], load_sems.at[k])
                    else:
                        rloads[k].wait()
                        y = acc + outstage[k, 1, :, :]
                        rms = jnp.sqrt(
                            jnp.mean(y * y, axis=-1, keepdims=True)
                            + 1e-6)
                        o = y / rms * gamma_ref[...]
                        sendb[k, 1, :, :] = o.astype(jnp.bfloat16)
                        rdma_cur[k] = start_hop(t + 1, sendb.at[k, 1],
                                                st)
                        wait_pending(st, 0)
                        outstage[k, 0, :, :] = o
                        store_from_slot(
                            st["f"] * R + p * R_SUB + st["off"], st, 0)

            for h in range(N_DEV - 1):
                t = t0 + (N_DEV - 1) + h
                slot = (h + 1) % 2
                for st in streams:
                    k = st["k"]
                    rdma_cur[k].wait()
                    if h < N_DEV - 2:
                        rdma_cur[k] = start_hop(
                            t + 1, recvb.at[k, t % NSLOT], st)
                    wait_pending(st, slot)
                    outstage[k, slot, :, :] = (
                        recvb[k, t % NSLOT, :, :].astype(jnp.float32))
                    store_from_slot(
                        st["g"](h) * R + p * R_SUB + st["off"], st,
                        slot)

        for st in streams:
            for slot in (0, 1):
                wait_pending(st, slot)

    return pl.pallas_call(
        body,
        out_shape=jax.ShapeDtypeStruct((M, D), jnp.float32),
        in_specs=[
            pl.BlockSpec(memory_space=pl.ANY),
            pl.BlockSpec(memory_space=pl.ANY),
            pl.BlockSpec(memory_space=pltpu.VMEM),
        ],
        out_specs=pl.BlockSpec(memory_space=pl.ANY),
        scratch_shapes=[
            pltpu.VMEM((NSTR, H, D), jnp.float32),
            pltpu.VMEM((NSTR, 2, H, D), jnp.bfloat16),
            pltpu.VMEM((NSTR, NSLOT, H, D), jnp.bfloat16),
            pltpu.VMEM((NSTR, 2, H, D), jnp.float32),
            pltpu.SemaphoreType.DMA((NSTR,)),
            pltpu.SemaphoreType.DMA((NSTR,)),
            pltpu.SemaphoreType.DMA((NSTR * 2,)),
            pltpu.SemaphoreType.DMA((NSTR * 2,)),
            pltpu.SemaphoreType.DMA((NSTR * NSLOT,)),
        ],
        compiler_params=pltpu.CompilerParams(
            collective_id=0,
            vmem_limit_bytes=63 * 1024 * 1024,
        ),
    )(partial, resid, gamma2)


# device time: 323254 ns/iter; 2.0454x vs baseline; 1.0137x over previous
import jax
import jax.numpy as jnp
from jax import lax
from jax.experimental import pallas as pl
from jax.experimental.pallas import tpu as pltpu

N_DEV = 4
R = 2048
R_SUB = 1024
NSTR = 4
H = R_SUB // NSTR
P = R // R_SUB
D = 2048
NSLOT = 5


def kernel(partial, resid, gamma):
    M = resid.shape[0]
    assert partial.shape == (1, M, D) and M == N_DEV * R

    gamma2 = gamma.reshape(1, D)

    def body(partial_ref, resid_ref, gamma_ref, out_ref,
             lstage, sendb, recvb, outstage,
             load_sems, resid_sems, store_sems, send_sems, recv_sems):
        i = lax.axis_index("i")
        left = (i + N_DEV - 1) % N_DEV
        right = (i + 1) % N_DEV

        barrier_sem = pltpu.get_barrier_semaphore()
        for nbr in (left, right):
            pl.semaphore_signal(
                barrier_sem, inc=1,
                device_id=(nbr,), device_id_type=pl.DeviceIdType.MESH,
            )
        pl.semaphore_wait(barrier_sem, 2)

        ring_a = dict(dst=right,
                      c_recv=lambda s: (i + N_DEV - s - 1) % N_DEV,
                      f=(i + 1) % N_DEV,
                      g=lambda h: (i + N_DEV - h) % N_DEV)
        ring_b = dict(dst=left,
                      c_recv=lambda s: (i + s + 1) % N_DEV,
                      f=(i + N_DEV - 1) % N_DEV,
                      g=lambda h: (i + h) % N_DEV)
        streams = [
            dict(k=0, off=0 * H, **ring_a, pending=[None, None]),
            dict(k=1, off=2 * H, **ring_b, pending=[None, None]),
            dict(k=2, off=1 * H, **ring_a, pending=[None, None]),
            dict(k=3, off=3 * H, **ring_b, pending=[None, None]),
        ]

        def start_load(ref, c, p, st, dst, sem):
            cp = pltpu.make_async_copy(
                ref.at[pl.ds(c * R + p * R_SUB + st["off"], H), :],
                dst, sem)
            cp.start()
            return cp

        def start_hop(t, src, st):
            k = st["k"]
            rdma = pltpu.make_async_remote_copy(
                src_ref=src,
                dst_ref=recvb.at[k, t % NSLOT],
                send_sem=send_sems.at[k * 2 + t % 2],
                recv_sem=recv_sems.at[k * NSLOT + t % NSLOT],
                device_id=(st["dst"],),
                device_id_type=pl.DeviceIdType.MESH,
            )
            rdma.start()
            return rdma

        def wait_pending(st, slot):
            if st["pending"][slot] is not None:
                st["pending"][slot].wait()
                st["pending"][slot] = None

        def store_from_slot(row0, st, slot):
            wait_pending(st, slot)
            k = st["k"]
            s_ = pltpu.make_async_copy(
                outstage.at[k, slot], out_ref.at[pl.ds(row0, H), :],
                store_sems.at[k * 2 + slot])
            s_.start()
            st["pending"][slot] = s_

        rdma_cur = [None] * NSTR
        rdma_nextpass = [None] * NSTR
        loads = [None] * NSTR
        for p in range(P):
            t0 = p * 2 * (N_DEV - 1)

            rloads = [None] * NSTR
            if p == 0:
                for st in streams:
                    k = st["k"]
                    loads[k] = start_load(partial_ref.at[0], i, p, st,
                                          lstage.at[k],
                                          load_sems.at[k])
                for st in streams:
                    k = st["k"]
                    loads[k].wait()
                    sendb[k, 0, :, :] = (
                        lstage[k, :, :].astype(jnp.bfloat16))
                    rdma_cur[k] = start_hop(t0, sendb.at[k, 0], st)
                    loads[k] = start_load(partial_ref.at[0],
                                          st["c_recv"](0), p, st,
                                          lstage.at[k],
                                          load_sems.at[k])
                    wait_pending(st, 1)
                    rloads[k] = start_load(resid_ref, st["f"], p, st,
                                           outstage.at[k, 1],
                                           resid_sems.at[k])
            else:
                for st in streams:
                    k = st["k"]
                    rdma_cur[k] = rdma_nextpass[k]
                    wait_pending(st, 1)
                    rloads[k] = start_load(resid_ref, st["f"], p, st,
                                           outstage.at[k, 1],
                                           resid_sems.at[k])

            for s in range(N_DEV - 1):
                t = t0 + s
                for st in streams:
                    k = st["k"]
                    loads[k].wait()
                    rdma_cur[k].wait()
                    acc = (recvb[k, t % NSLOT, :, :]
                           .astype(jnp.float32) + lstage[k, :, :])
                    if s < N_DEV - 2:
                        sendb[k, (s + 1) % 2, :, :] = (
                            acc.astype(jnp.bfloat16))
                        rdma_cur[k] = start_hop(
                            t + 1, sendb.at[k, (s + 1) % 2], st)
                        loads[k] = start_load(
                            partial_ref.at[0], st["c_recv"](s + 1), p,
                            st, lstage.at[k], load_sems.at[k])
                    else:
                        rloads[k].wait()
                        y = acc + outstage[k, 1, :, :]
                        rms = jnp.sqrt(
                            jnp.mean(y * y, axis=-1, keepdims=True)
                            + 1e-6)
                        o = y / rms * gamma_ref[...]
                        sendb[k, 1, :, :] = o.astype(jnp.bfloat16)
                        rdma_cur[k] = start_hop(t + 1, sendb.at[k, 1],
                                                st)
                        wait_pending(st, 0)
                        outstage[k, 0, :, :] = o
                        store_from_slot(
                            st["f"] * R + p * R_SUB + st["off"], st, 0)
                        if p < P - 1:
                            loads[k] = start_load(
                                partial_ref.at[0], i, p + 1, st,
                                lstage.at[k], load_sems.at[k])

            for h in range(N_DEV - 1):
                t = t0 + (N_DEV - 1) + h
                slot = (h + 1) % 2
                for st in streams:
                    k = st["k"]
                    rdma_cur[k].wait()
                    if h < N_DEV - 2:
                        rdma_cur[k] = start_hop(
                            t + 1, recvb.at[k, t % NSLOT], st)
                    if h == 1 and p < P - 1:
                        loads[k].wait()
                        sendb[k, 0, :, :] = (
                            lstage[k, :, :].astype(jnp.bfloat16))
                        rdma_nextpass[k] = start_hop(
                            t0 + 6, sendb.at[k, 0], st)
                        loads[k] = start_load(partial_ref.at[0],
                                              st["c_recv"](0), p + 1,
                                              st, lstage.at[k],
                                              load_sems.at[k])
                    wait_pending(st, slot)
                    outstage[k, slot, :, :] = (
                        recvb[k, t % NSLOT, :, :].astype(jnp.float32))
                    store_from_slot(
                        st["g"](h) * R + p * R_SUB + st["off"], st,
                        slot)

        for st in streams:
            for slot in (0, 1):
                wait_pending(st, slot)

    return pl.pallas_call(
        body,
        out_shape=jax.ShapeDtypeStruct((M, D), jnp.float32),
        in_specs=[
            pl.BlockSpec(memory_space=pl.ANY),
            pl.BlockSpec(memory_space=pl.ANY),
            pl.BlockSpec(memory_space=pltpu.VMEM),
        ],
        out_specs=pl.BlockSpec(memory_space=pltpu.MemorySpace.HBM),
        scratch_shapes=[
            pltpu.VMEM((NSTR, H, D), jnp.float32),
            pltpu.VMEM((NSTR, 2, H, D), jnp.bfloat16),
            pltpu.VMEM((NSTR, NSLOT, H, D), jnp.bfloat16),
            pltpu.VMEM((NSTR, 2, H, D), jnp.float32),
            pltpu.SemaphoreType.DMA((NSTR,)),
            pltpu.SemaphoreType.DMA((NSTR,)),
            pltpu.SemaphoreType.DMA((NSTR * 2,)),
            pltpu.SemaphoreType.DMA((NSTR * 2,)),
            pltpu.SemaphoreType.DMA((NSTR * NSLOT,)),
        ],
        compiler_params=pltpu.CompilerParams(
            collective_id=0,
            vmem_limit_bytes=63 * 1024 * 1024,
        ),
    )(partial, resid, gamma2)
